# Initial kernel scaffold; baseline (speedup 1.0000x reference)
import jax
import jax.numpy as jnp
from jax.experimental import pallas as pl


def kernel(z, edge_index, edge_type, rel_emb):
    raise NotImplementedError("write your pallas kernel here")


# speedup vs baseline: 2.1770x; 2.1770x over previous
import functools

import jax
import jax.numpy as jnp
from jax import lax
from jax.experimental import pallas as pl
from jax.experimental.pallas import tpu as pltpu
from jax.experimental.pallas import tpu_sc as plsc

NC = 2
NS = 16
NW = NC * NS
L = 16
C = 80


def _make_kernel(E, H):
  n_per_w = E // NW
  n_chunks = n_per_w // C
  groups = C // L
  mesh = plsc.VectorSubcoreMesh(core_axis_name="c", subcore_axis_name="s")

  @functools.partial(
      pl.kernel,
      out_type=jax.ShapeDtypeStruct((E,), jnp.float32),
      mesh=mesh,
      compiler_params=pltpu.CompilerParams(needs_layout_passes=False),
      scratch_types=[
          pltpu.VMEM((C,), jnp.int32),
          pltpu.VMEM((C,), jnp.int32),
          pltpu.VMEM((C,), jnp.int32),
          pltpu.VMEM((C, H), jnp.float32),
          pltpu.VMEM((C, H), jnp.float32),
          pltpu.VMEM((C, H), jnp.float32),
          pltpu.VMEM((C,), jnp.float32),
          pltpu.SemaphoreType.DMA,
          pltpu.SemaphoreType.DMA,
          pltpu.SemaphoreType.DMA,
      ],
  )
  def distmult(z_hbm, src_hbm, dst_hbm, et_hbm, rel_hbm, out_hbm,
               si, di, ti, sr, dr, rr, ov, sem_s, sem_d, sem_r):
    wid = lax.axis_index("s") * NC + lax.axis_index("c")
    wbase = wid * n_per_w
    row_idx = lax.iota(jnp.int32, L)

    def chunk(i, _):
      base = wbase + i * C
      pltpu.sync_copy(src_hbm.at[pl.ds(base, C)], si)
      pltpu.sync_copy(dst_hbm.at[pl.ds(base, C)], di)
      pltpu.sync_copy(et_hbm.at[pl.ds(base, C)], ti)
      cs = pltpu.async_copy(z_hbm.at[si], sr, sem_s)
      cd = pltpu.async_copy(z_hbm.at[di], dr, sem_d)
      cr = pltpu.async_copy(rel_hbm.at[ti], rr, sem_r)
      cs.wait()
      cd.wait()
      cr.wait()

      def group(g, _):
        e0 = g * L
        res = jnp.zeros((L,), jnp.float32)
        for el in range(L):
          e = e0 + el
          acc = (sr[e, pl.ds(0, L)] * dr[e, pl.ds(0, L)]) * rr[e, pl.ds(0, L)]
          for j in range(1, H // L):
            o = j * L
            acc = acc + (sr[e, pl.ds(o, L)] * dr[e, pl.ds(o, L)]) * rr[e, pl.ds(o, L)]
          total = jnp.sum(acc)
          res = jnp.where(row_idx == el, jnp.full((L,), total), res)
        ov[pl.ds(e0, L)] = res
        return _

      lax.fori_loop(0, groups, group, None)
      pltpu.sync_copy(ov, out_hbm.at[pl.ds(base, C)])
      return _

    lax.fori_loop(0, n_chunks, chunk, None)

  return distmult


def kernel(z, edge_index, edge_type, rel_emb):
  E = edge_index.shape[1]
  H = z.shape[1]
  src = edge_index[0].astype(jnp.int32)
  dst = edge_index[1].astype(jnp.int32)
  et = edge_type.astype(jnp.int32)
  return _make_kernel(E, H)(z, src, dst, et, rel_emb)


# speedup vs baseline: 3.1933x; 1.4669x over previous
import functools

import jax
import jax.numpy as jnp
from jax import lax
from jax.experimental import pallas as pl
from jax.experimental.pallas import tpu as pltpu
from jax.experimental.pallas import tpu_sc as plsc

NC = 2
NS = 16
NW = NC * NS
L = 16
C = 80


def _make_kernel(E, H):
  n_per_w = E // NW
  n_chunks = n_per_w // C
  groups = C // L
  mesh = plsc.VectorSubcoreMesh(core_axis_name="c", subcore_axis_name="s")

  @functools.partial(
      pl.kernel,
      out_type=jax.ShapeDtypeStruct((E,), jnp.float32),
      mesh=mesh,
      compiler_params=pltpu.CompilerParams(needs_layout_passes=False),
      scratch_types=[
          pltpu.VMEM((n_per_w,), jnp.int32),
          pltpu.VMEM((n_per_w,), jnp.int32),
          pltpu.VMEM((n_per_w,), jnp.int32),
          pltpu.VMEM((C, H), jnp.float32),
          pltpu.VMEM((C, H), jnp.float32),
          pltpu.VMEM((C, H), jnp.float32),
          pltpu.VMEM((C, H), jnp.float32),
          pltpu.VMEM((C, H), jnp.float32),
          pltpu.VMEM((C, H), jnp.float32),
          pltpu.VMEM((n_per_w,), jnp.float32),
          pltpu.SemaphoreType.DMA,
          pltpu.SemaphoreType.DMA,
          pltpu.SemaphoreType.DMA,
          pltpu.SemaphoreType.DMA,
          pltpu.SemaphoreType.DMA,
          pltpu.SemaphoreType.DMA,
      ],
  )
  def distmult(z_hbm, src_hbm, dst_hbm, et_hbm, rel_hbm, out_hbm,
               si, di, ti, sr0, dr0, rr0, sr1, dr1, rr1, ov,
               ss0, sd0, st0, ss1, sd1, st1):
    wid = lax.axis_index("s") * NC + lax.axis_index("c")
    wbase = wid * n_per_w
    row_idx = lax.iota(jnp.int32, L)

    pltpu.sync_copy(src_hbm.at[pl.ds(wbase, n_per_w)], si)
    pltpu.sync_copy(dst_hbm.at[pl.ds(wbase, n_per_w)], di)
    pltpu.sync_copy(et_hbm.at[pl.ds(wbase, n_per_w)], ti)

    bufs = ((sr0, dr0, rr0, ss0, sd0, st0),
            (sr1, dr1, rr1, ss1, sd1, st1))

    def copies(c, b):
      sr, dr, rr, ss, sd, st = bufs[b]
      off = c * C
      return (
          pltpu.make_async_copy(z_hbm.at[si.at[pl.ds(off, C)]], sr, ss),
          pltpu.make_async_copy(z_hbm.at[di.at[pl.ds(off, C)]], dr, sd),
          pltpu.make_async_copy(rel_hbm.at[ti.at[pl.ds(off, C)]], rr, st),
      )

    def issue(c, b):
      @pl.when(c < n_chunks)
      def _():
        for cp in copies(c, b):
          cp.start()

    def compute(c, b):
      sr, dr, rr, _, _, _ = bufs[b]
      for cp in copies(c, b):
        cp.wait()

      def group(g, _):
        e0 = g * L
        res = jnp.zeros((L,), jnp.float32)
        for el in range(L):
          e = e0 + el
          acc = (sr[e, pl.ds(0, L)] * dr[e, pl.ds(0, L)]) * rr[e, pl.ds(0, L)]
          for j in range(1, H // L):
            o = j * L
            acc = acc + (sr[e, pl.ds(o, L)] * dr[e, pl.ds(o, L)]) * rr[e, pl.ds(o, L)]
          total = jnp.sum(acc)
          res = jnp.where(row_idx == el, jnp.full((L,), total), res)
        ov[pl.ds(c * C + g * L, L)] = res
        return _

      lax.fori_loop(0, groups, group, None)

    issue(0, 0)

    def pair(j, _):
      c0 = 2 * j
      c1 = c0 + 1
      issue(c1, 1)
      compute(c0, 0)
      issue(c0 + 2, 0)

      @pl.when(c1 < n_chunks)
      def _():
        compute(c1, 1)

      return _

    lax.fori_loop(0, (n_chunks + 1) // 2, pair, None)
    pltpu.sync_copy(ov, out_hbm.at[pl.ds(wbase, n_per_w)])

  return distmult


def kernel(z, edge_index, edge_type, rel_emb):
  E = edge_index.shape[1]
  H = z.shape[1]
  src = edge_index[0].astype(jnp.int32)
  dst = edge_index[1].astype(jnp.int32)
  et = edge_type.astype(jnp.int32)
  return _make_kernel(E, H)(z, src, dst, et, rel_emb)


# speedup vs baseline: 7.9538x; 2.4907x over previous
import functools

import jax
import jax.numpy as jnp
from jax import lax
from jax.experimental import pallas as pl
from jax.experimental.pallas import tpu as pltpu
from jax.experimental.pallas import tpu_sc as plsc

NC = 2
NS = 16
NW = NC * NS
L = 16
C = 80


def _make_kernel(E, H):
  n_per_w = E // NW
  n_chunks = n_per_w // C
  groups = C // L
  mesh = plsc.VectorSubcoreMesh(core_axis_name="c", subcore_axis_name="s")

  @functools.partial(
      pl.kernel,
      out_type=jax.ShapeDtypeStruct((E,), jnp.float32),
      mesh=mesh,
      compiler_params=pltpu.CompilerParams(needs_layout_passes=False),
      scratch_types=[
          pltpu.VMEM((n_per_w,), jnp.int32),
          pltpu.VMEM((n_per_w,), jnp.int32),
          pltpu.VMEM((n_per_w,), jnp.int32),
          pltpu.VMEM((C, H), jnp.float32),
          pltpu.VMEM((C, H), jnp.float32),
          pltpu.VMEM((C, H), jnp.float32),
          pltpu.VMEM((C, H), jnp.float32),
          pltpu.VMEM((C, H), jnp.float32),
          pltpu.VMEM((C, H), jnp.float32),
          pltpu.VMEM((n_per_w + L,), jnp.float32),
          pltpu.SemaphoreType.DMA,
          pltpu.SemaphoreType.DMA,
          pltpu.SemaphoreType.DMA,
          pltpu.SemaphoreType.DMA,
          pltpu.SemaphoreType.DMA,
          pltpu.SemaphoreType.DMA,
      ],
  )
  def distmult(z_hbm, src_hbm, dst_hbm, et_hbm, rel_hbm, out_hbm,
               si, di, ti, sr0, dr0, rr0, sr1, dr1, rr1, ov,
               ss0, sd0, st0, ss1, sd1, st1):
    wid = lax.axis_index("s") * NC + lax.axis_index("c")
    wbase = wid * n_per_w
    mask_last = lax.iota(jnp.int32, L) == (L - 1)

    pltpu.sync_copy(src_hbm.at[pl.ds(wbase, n_per_w)], si)
    pltpu.sync_copy(dst_hbm.at[pl.ds(wbase, n_per_w)], di)
    pltpu.sync_copy(et_hbm.at[pl.ds(wbase, n_per_w)], ti)

    bufs = ((sr0, dr0, rr0, ss0, sd0, st0),
            (sr1, dr1, rr1, ss1, sd1, st1))

    def copies(c, b):
      sr, dr, rr, ss, sd, st = bufs[b]
      off = c * C
      return (
          pltpu.make_async_copy(z_hbm.at[si.at[pl.ds(off, C)]], sr, ss),
          pltpu.make_async_copy(z_hbm.at[di.at[pl.ds(off, C)]], dr, sd),
          pltpu.make_async_copy(rel_hbm.at[ti.at[pl.ds(off, C)]], rr, st),
      )

    def issue(c, b):
      @pl.when(c < n_chunks)
      def _():
        for cp in copies(c, b):
          cp.start()

    def compute(c, b):
      sr, dr, rr, _, _, _ = bufs[b]
      for cp in copies(c, b):
        cp.wait()

      def group(g, _):
        e0 = g * L
        for el in range(L):
          e = e0 + el
          acc = (sr[e, pl.ds(0, L)] * dr[e, pl.ds(0, L)]) * rr[e, pl.ds(0, L)]
          for j in range(1, H // L):
            o = j * L
            acc = acc + (sr[e, pl.ds(o, L)] * dr[e, pl.ds(o, L)]) * rr[e, pl.ds(o, L)]
          plsc.store_compressed(ov.at[pl.ds(c * C + e, L)], plsc.cumsum(acc),
                                mask=mask_last)
        return _

      lax.fori_loop(0, groups, group, None)

    issue(0, 0)

    def pair(j, _):
      c0 = 2 * j
      c1 = c0 + 1
      issue(c1, 1)
      compute(c0, 0)
      issue(c0 + 2, 0)

      @pl.when(c1 < n_chunks)
      def _():
        compute(c1, 1)

      return _

    lax.fori_loop(0, (n_chunks + 1) // 2, pair, None)
    pltpu.sync_copy(ov.at[pl.ds(0, n_per_w)], out_hbm.at[pl.ds(wbase, n_per_w)])

  return distmult


def kernel(z, edge_index, edge_type, rel_emb):
  E = edge_index.shape[1]
  H = z.shape[1]
  src = edge_index[0].astype(jnp.int32)
  dst = edge_index[1].astype(jnp.int32)
  et = edge_type.astype(jnp.int32)
  return _make_kernel(E, H)(z, src, dst, et, rel_emb)
